# baseline (speedup 1.0000x reference)
import functools

import jax
import jax.numpy as jnp
from jax import lax
from jax.experimental import pallas as pl
from jax.experimental.pallas import tpu as pltpu
from jax.experimental.pallas import tpu_sc as plsc

_ELECTRODE_POS = (
    (1, 0), (2, 8), (3, 9), (4, 17), (5, 18), (6, 26),
    (7, 21), (8, 23), (9, 30), (10, 31), (11, 32), (12, 39),
    (13, 40), (14, 41), (15, 48), (16, 49), (17, 50),
)
_POS_TO_ELEC = {p: e for e, p in _ELECTRODE_POS}

_NPOS = 54
_BATCH = 64
_ROWS = _NPOS * _BATCH
_H, _W = 32, 512
_NW = 32
_RPW = _ROWS // _NW


def _sc_body(x_hbm, rem_hbm, zeros_hbm, out_hbm, rem_v, buf, zbuf, flags):
    wid = lax.axis_index("s") * 2 + lax.axis_index("c")

    pltpu.sync_copy(rem_hbm, rem_v)
    pltpu.sync_copy(zeros_hbm, zbuf)

    r0 = rem_v[pl.ds(0, 16)]
    r1 = rem_v[pl.ds(16, 16)]
    rem_vals = [r0[j] for j in range(16)] + [r1[0], r1[1]]

    for p in range(_NPOS):
        e = _POS_TO_ELEC.get(p)
        if e is None:
            flags[p] = jnp.int32(0)
        else:
            pres = rem_vals[0] == e
            for j in range(1, 18):
                pres = jnp.logical_or(pres, rem_vals[j] == e)
            flags[p] = pres.astype(jnp.int32)

    base = wid * _RPW

    def body(i, carry):
        r = base + i
        zf = flags[lax.rem(r, _NPOS)]

        @pl.when(zf == 0)
        def _():
            pltpu.sync_copy(x_hbm.at[r], buf)
            pltpu.sync_copy(buf, out_hbm.at[r])

        @pl.when(zf != 0)
        def _():
            pltpu.sync_copy(zbuf, out_hbm.at[r])

        return carry

    lax.fori_loop(0, _RPW, body, jnp.int32(0))


def kernel(x, removed_electrodes):
    xt = jnp.transpose(x, (0, 3, 4, 1, 2)).reshape(_ROWS, _H, _W)
    rem = jnp.zeros((32,), jnp.int32).at[:18].set(
        removed_electrodes.astype(jnp.int32))
    zeros = jnp.zeros((_H, _W), jnp.float32)
    mesh = plsc.VectorSubcoreMesh(core_axis_name="c", subcore_axis_name="s")
    sck = functools.partial(
        pl.kernel,
        out_type=jax.ShapeDtypeStruct((_ROWS, _H, _W), jnp.float32),
        mesh=mesh,
        scratch_types=[
            pltpu.VMEM((32,), jnp.int32),
            pltpu.VMEM((_H, _W), jnp.float32),
            pltpu.VMEM((_H, _W), jnp.float32),
            pltpu.SMEM((_NPOS,), jnp.int32),
        ],
    )(_sc_body)
    out = sck(xt, rem, zeros)
    return jnp.transpose(out.reshape(_BATCH, 6, 9, _H, _W), (0, 3, 4, 1, 2))


# speedup vs baseline: 1.2727x; 1.2727x over previous
import functools

import jax
import jax.numpy as jnp
from jax import lax
from jax.experimental import pallas as pl
from jax.experimental.pallas import tpu as pltpu
from jax.experimental.pallas import tpu_sc as plsc

_ELECTRODE_POS = (
    (1, 0), (2, 8), (3, 9), (4, 17), (5, 18), (6, 26),
    (7, 21), (8, 23), (9, 30), (10, 31), (11, 32), (12, 39),
    (13, 40), (14, 41), (15, 48), (16, 49), (17, 50),
)
_POS_TO_ELEC = {p: e for e, p in _ELECTRODE_POS}

_NPOS = 54
_BATCH = 64
_ROWS = _NPOS * _BATCH
_H, _W = 32, 512
_NW = 32
_RPW = _ROWS // _NW


def _sc_body(x_hbm, rem_hbm, zeros_hbm, out_hbm,
             rem_v, buf0, buf1, zbuf, ksm, zsm,
             rs0, rs1, ws0, ws1, zsem):
    wid = lax.axis_index("s") * 2 + lax.axis_index("c")
    base = wid * _RPW

    pltpu.sync_copy(rem_hbm, rem_v)
    pltpu.sync_copy(zeros_hbm, zbuf)

    r0v = rem_v[pl.ds(0, 16)]
    r1v = rem_v[pl.ds(16, 16)]
    rem_vals = [r0v[j] for j in range(16)] + [r1v[0], r1v[1]]

    pres = {}
    for e, _p in _ELECTRODE_POS:
        acc = rem_vals[0] == e
        for j in range(1, 18):
            acc = jnp.logical_or(acc, rem_vals[j] == e)
        pres[e] = acc

    nk = jnp.int32(0)
    nz = jnp.int32(0)
    for i in range(_RPW):
        e = _POS_TO_ELEC.get(i % _NPOS)
        if e is None:
            ksm[nk] = jnp.int32(i)
            nk = nk + 1
        else:
            z = pres[e].astype(jnp.int32)
            ksm[nk] = jnp.int32(i)
            zsm[nz] = jnp.int32(i)
            nk = nk + (1 - z)
            nz = nz + z

    def zfire(j, carry):
        r = base + zsm[j]
        pltpu.make_async_copy(zbuf, out_hbm.at[r], zsem).start()
        return carry

    lax.fori_loop(0, nz, zfire, jnp.int32(0))

    bufs = (buf0, buf1)
    rsems = (rs0, rs1)
    wsems = (ws0, ws1)

    @pl.when(nk > 0)
    def _():
        pltpu.make_async_copy(x_hbm.at[base + ksm[0]], buf0, rs0).start()

    def kbody(j, carry):
        for b in range(2):
            opp = 1 - b

            @pl.when(lax.rem(j, 2) == b)
            def _():
                @pl.when(j + 1 < nk)
                def _():
                    @pl.when(j >= 1)
                    def _():
                        pltpu.make_async_copy(
                            bufs[opp], out_hbm.at[base], wsems[opp]).wait()

                    pltpu.make_async_copy(
                        x_hbm.at[base + ksm[j + 1]], bufs[opp],
                        rsems[opp]).start()

                pltpu.make_async_copy(
                    x_hbm.at[base], bufs[b], rsems[b]).wait()
                pltpu.make_async_copy(
                    bufs[b], out_hbm.at[base + ksm[j]], wsems[b]).start()

        return carry

    lax.fori_loop(0, nk, kbody, jnp.int32(0))

    @pl.when(nk > 0)
    def _():
        last = lax.rem(nk - 1, 2)
        for b in range(2):
            @pl.when(last == b)
            def _():
                pltpu.make_async_copy(
                    bufs[b], out_hbm.at[base], wsems[b]).wait()

                @pl.when(nk > 1)
                def _():
                    pltpu.make_async_copy(
                        bufs[1 - b], out_hbm.at[base], wsems[1 - b]).wait()

    def zdrain(j, carry):
        pltpu.make_async_copy(zbuf, out_hbm.at[base], zsem).wait()
        return carry

    lax.fori_loop(0, nz, zdrain, jnp.int32(0))


def kernel(x, removed_electrodes):
    xt = jnp.transpose(x, (0, 3, 4, 1, 2)).reshape(_ROWS, _H, _W)
    rem = jnp.zeros((32,), jnp.int32).at[:18].set(
        removed_electrodes.astype(jnp.int32))
    zeros = jnp.zeros((_H, _W), jnp.float32)
    mesh = plsc.VectorSubcoreMesh(core_axis_name="c", subcore_axis_name="s")
    sck = functools.partial(
        pl.kernel,
        out_type=jax.ShapeDtypeStruct((_ROWS, _H, _W), jnp.float32),
        mesh=mesh,
        scratch_types=[
            pltpu.VMEM((32,), jnp.int32),
            pltpu.VMEM((_H, _W), jnp.float32),
            pltpu.VMEM((_H, _W), jnp.float32),
            pltpu.VMEM((_H, _W), jnp.float32),
            pltpu.SMEM((_RPW + 1,), jnp.int32),
            pltpu.SMEM((_RPW + 1,), jnp.int32),
            pltpu.SemaphoreType.DMA,
            pltpu.SemaphoreType.DMA,
            pltpu.SemaphoreType.DMA,
            pltpu.SemaphoreType.DMA,
            pltpu.SemaphoreType.DMA,
        ],
    )(_sc_body)
    out = sck(xt, rem, zeros)
    return jnp.transpose(out.reshape(_BATCH, 6, 9, _H, _W), (0, 3, 4, 1, 2))


# speedup vs baseline: 1.3219x; 1.0386x over previous
import functools

import jax
import jax.numpy as jnp
from jax import lax
from jax.experimental import pallas as pl
from jax.experimental.pallas import tpu as pltpu
from jax.experimental.pallas import tpu_sc as plsc

_ELECTRODE_POS = (
    (1, 0), (2, 8), (3, 9), (4, 17), (5, 18), (6, 26),
    (7, 21), (8, 23), (9, 30), (10, 31), (11, 32), (12, 39),
    (13, 40), (14, 41), (15, 48), (16, 49), (17, 50),
)
_POS_TO_ELEC = {p: e for e, p in _ELECTRODE_POS}

_NPOS = 54
_BATCH = 64
_ROWS = _NPOS * _BATCH
_H, _W = 32, 512
_NW = 32
_RPW = _ROWS // _NW
_DEPTH = 6
_AHEAD = 3


def _sc_body(x_hbm, rem_hbm, zeros_hbm, out_hbm, rem_v, zbuf, *rest):
    bufs = rest[:_DEPTH]
    ksm, zsm = rest[_DEPTH:_DEPTH + 2]
    rsems = rest[_DEPTH + 2:2 * _DEPTH + 2]
    wsems = rest[2 * _DEPTH + 2:3 * _DEPTH + 2]
    zsem = rest[3 * _DEPTH + 2]

    wid = lax.axis_index("s") * 2 + lax.axis_index("c")
    base = wid * _RPW

    pltpu.sync_copy(rem_hbm, rem_v)
    pltpu.sync_copy(zeros_hbm, zbuf)

    r0v = rem_v[pl.ds(0, 16)]
    r1v = rem_v[pl.ds(16, 16)]
    rem_vals = [r0v[j] for j in range(16)] + [r1v[0], r1v[1]]

    pres = {}
    for e, _p in _ELECTRODE_POS:
        acc = rem_vals[0] == e
        for j in range(1, 18):
            acc = jnp.logical_or(acc, rem_vals[j] == e)
        pres[e] = acc

    nk = jnp.int32(0)
    nz = jnp.int32(0)
    for i in range(_RPW):
        e = _POS_TO_ELEC.get(i % _NPOS)
        if e is None:
            ksm[nk] = jnp.int32(i)
            nk = nk + 1
        else:
            z = pres[e].astype(jnp.int32)
            ksm[nk] = jnp.int32(i)
            zsm[nz] = jnp.int32(i)
            nk = nk + (1 - z)
            nz = nz + z

    def zfire(j, carry):
        r = base + zsm[j]
        pltpu.make_async_copy(zbuf, out_hbm.at[r], zsem).start()
        return carry

    lax.fori_loop(0, nz, zfire, jnp.int32(0))

    for k in range(_AHEAD):
        @pl.when(k < nk)
        def _(k=k):
            pltpu.make_async_copy(
                x_hbm.at[base + ksm[k]], bufs[k], rsems[k]).start()

    def kbody(j, carry):
        for b in range(_DEPTH):
            @pl.when(lax.rem(j, _DEPTH) == b)
            def _(b=b):
                pltpu.make_async_copy(
                    x_hbm.at[base], bufs[b], rsems[b]).wait()
                pltpu.make_async_copy(
                    bufs[b], out_hbm.at[base + ksm[j]], wsems[b]).start()

                c = (b + _AHEAD) % _DEPTH

                @pl.when(j + _AHEAD < nk)
                def _():
                    @pl.when(j + _AHEAD >= _DEPTH)
                    def _():
                        pltpu.make_async_copy(
                            bufs[c], out_hbm.at[base], wsems[c]).wait()

                    pltpu.make_async_copy(
                        x_hbm.at[base + ksm[j + _AHEAD]], bufs[c],
                        rsems[c]).start()

        return carry

    lax.fori_loop(0, nk, kbody, jnp.int32(0))

    for b in range(_DEPTH):
        @pl.when(b < nk)
        def _(b=b):
            pltpu.make_async_copy(bufs[b], out_hbm.at[base], wsems[b]).wait()

    def zdrain(j, carry):
        pltpu.make_async_copy(zbuf, out_hbm.at[base], zsem).wait()
        return carry

    lax.fori_loop(0, nz, zdrain, jnp.int32(0))


def kernel(x, removed_electrodes):
    xt = jnp.transpose(x, (0, 3, 4, 1, 2)).reshape(_ROWS, _H, _W)
    rem = jnp.zeros((32,), jnp.int32).at[:18].set(
        removed_electrodes.astype(jnp.int32))
    zeros = jnp.zeros((_H, _W), jnp.float32)
    mesh = plsc.VectorSubcoreMesh(core_axis_name="c", subcore_axis_name="s")
    sck = functools.partial(
        pl.kernel,
        out_type=jax.ShapeDtypeStruct((_ROWS, _H, _W), jnp.float32),
        mesh=mesh,
        scratch_types=(
            [pltpu.VMEM((32,), jnp.int32),
             pltpu.VMEM((_H, _W), jnp.float32)]
            + [pltpu.VMEM((_H, _W), jnp.float32) for _ in range(_DEPTH)]
            + [pltpu.SMEM((_RPW + 1,), jnp.int32),
               pltpu.SMEM((_RPW + 1,), jnp.int32)]
            + [pltpu.SemaphoreType.DMA for _ in range(2 * _DEPTH + 1)]
        ),
    )(_sc_body)
    out = sck(xt, rem, zeros)
    return jnp.transpose(out.reshape(_BATCH, 6, 9, _H, _W), (0, 3, 4, 1, 2))
